# baseline (speedup 1.0000x reference)
import functools

import jax
import jax.numpy as jnp
from jax import lax
from jax.experimental import pallas as pl
from jax.experimental.pallas import tpu as pltpu
from jax.experimental.pallas import tpu_sc as plsc

_W = 8


@functools.cache
def _make_gather(V, D, B):
    info = plsc.get_sparse_core_info()
    NC, NS = info.num_cores, info.num_subcores
    NW = NC * NS
    assert B % (16 * NW) == 0, (V, D, B, NW)
    b_per_w = B // NW
    n_pairs = b_per_w // 16
    mesh = plsc.VectorSubcoreMesh(core_axis_name="c", subcore_axis_name="s")

    @functools.partial(
        pl.kernel,
        mesh=mesh,
        compiler_params=pltpu.CompilerParams(
            use_tc_tiling_on_sc=True, needs_layout_passes=False
        ),
        out_type=jax.ShapeDtypeStruct((D, B), jnp.float32),
        scratch_types=[
            pltpu.VMEM((b_per_w,), jnp.int32),
            pltpu.VMEM((D, _W * 128), jnp.float32),
            pltpu.VMEM((D, _W * 128), jnp.float32),
            pltpu.VMEM((D, b_per_w), jnp.float32),
            pltpu.SemaphoreType.DMA,
            pltpu.SemaphoreType.DMA,
        ],
    )
    def gather_kernel(
        table_hbm, idx_hbm, out_hbm, idx_v, buf_a, buf_b, cols_v, sem_a, sem_b
    ):
        wid = lax.axis_index("s") * NC + lax.axis_index("c")
        base = wid * b_per_w
        pltpu.sync_copy(idx_hbm.at[pl.ds(base, b_per_w)], idx_v)

        def issue(v, lo, buf, sem):
            for j in range(lo, lo + _W):
                tcol = pl.multiple_of((v[j] >> 7) << 7, 128)
                pltpu.async_copy(
                    table_hbm.at[:, pl.ds(tcol, 128)],
                    buf.at[:, pl.ds((j - lo) * 128, 128)],
                    sem,
                )

        def drain(buf, sem):
            pltpu.make_async_copy(
                table_hbm.at[:, pl.ds(0, _W * 128)], buf, sem
            ).wait()

        def extract(v, d, lo, buf):
            lane16 = lax.iota(jnp.int32, 16)
            mask = (lane16 >= lo) & (lane16 < lo + _W)
            src_col = ((lane16 - lo) & (_W - 1)) * 128 + (v & 127)
            dst_col = d * 16 + lane16
            for j in range(D):
                row = jnp.full((16,), j, jnp.int32)
                vals = plsc.load_gather(buf, [row, src_col], mask=mask)
                plsc.store_scatter(cols_v, [row, dst_col], vals, mask=mask)

        def pair(d):
            v = idx_v[pl.ds(d * 16, 16)]
            issue(v, 0, buf_a, sem_a)

            @pl.when(d > 0)
            def _():
                v_prev = idx_v[pl.ds((d - 1) * 16, 16)]
                drain(buf_b, sem_b)
                extract(v_prev, d - 1, _W, buf_b)

            drain(buf_a, sem_a)
            extract(v, d, 0, buf_a)
            issue(v, _W, buf_b, sem_b)

        pl.loop(0, n_pairs)(pair)
        v_last = idx_v[pl.ds((n_pairs - 1) * 16, 16)]
        drain(buf_b, sem_b)
        extract(v_last, n_pairs - 1, _W, buf_b)
        pltpu.sync_copy(cols_v, out_hbm.at[:, pl.ds(base, b_per_w)])

    return gather_kernel


def kernel(graph, feat, nodes, emb):
    V, D = emb.shape
    (B,) = nodes.shape
    out_t = _make_gather(V, D, B)(emb.T, nodes)
    return out_t.T
